# baseline (speedup 1.0000x reference)
import functools

import jax
import jax.numpy as jnp
from jax import lax
from jax.experimental import pallas as pl
from jax.experimental.pallas import tpu as pltpu
from jax.experimental.pallas import tpu_sc as plsc

_B = 16384
_D = 64
_H = 64
_N_CLASS = 128
_E_FACTOR = 1.0
_PACK = 128

_NC = 2
_NS = 16
_NW = _NC * _NS
_BPW = _B // _NW
_CH = 128
_NCH = _BPW // _CH


def _gather_body(ga_hbm, gb_hbm, pack_hbm, outa_hbm, outb_hbm,
                 idx_a, idx_b, rows_a, rows_b, sem, wsem):
    wid = lax.axis_index("s") * _NC + lax.axis_index("c")
    pltpu.sync_copy(ga_hbm.at[wid], idx_a)
    pltpu.sync_copy(gb_hbm.at[wid], idx_b)
    wcopies = []
    for j in range(_NCH):
        s = j % 2
        if j >= 2:
            wcopies.pop(0).wait()
            wcopies.pop(0).wait()
        ca = pltpu.async_copy(pack_hbm.at[idx_a.at[j]], rows_a.at[s], sem)
        cb = pltpu.async_copy(pack_hbm.at[idx_b.at[j]], rows_b.at[s], sem)
        ca.wait()
        cb.wait()
        base = (wid * _NCH + j) * _CH
        wcopies.append(pltpu.async_copy(rows_a.at[s], outa_hbm.at[pl.ds(base, _CH)], wsem))
        wcopies.append(pltpu.async_copy(rows_b.at[s], outb_hbm.at[pl.ds(base, _CH)], wsem))
    for cp in wcopies:
        cp.wait()


@jax.jit
def _sc_gather(ga, gb, packed):
    mesh = plsc.VectorSubcoreMesh(core_axis_name="c", subcore_axis_name="s")
    kern = functools.partial(
        pl.kernel,
        mesh=mesh,
        out_type=(
            jax.ShapeDtypeStruct((_B, _PACK), jnp.float32),
            jax.ShapeDtypeStruct((_B, _PACK), jnp.float32),
        ),
        scratch_types=[
            pltpu.VMEM((_NCH, _CH), jnp.int32),
            pltpu.VMEM((_NCH, _CH), jnp.int32),
            pltpu.VMEM((2, _CH, _PACK), jnp.float32),
            pltpu.VMEM((2, _CH, _PACK), jnp.float32),
            pltpu.SemaphoreType.DMA,
            pltpu.SemaphoreType.DMA,
        ],
    )(_gather_body)
    return kern(ga, gb, packed)


def _mlp_body(ra_ref, rb_ref, sa_ref, sb_ref, w1_ref, b1_ref, w2_ref, b2_ref,
              o_ref):
    sa = sa_ref[...] > 0
    sb = sb_ref[...] > 0
    ea = jnp.where(sa, ra_ref[:, _D:], ra_ref[:, :_D])
    eb = jnp.where(sb, rb_ref[:, _D:], rb_ref[:, :_D])
    e = (ea + eb) * (1.0 / _E_FACTOR)
    h = jnp.dot(e, w1_ref[...], preferred_element_type=jnp.float32) + b1_ref[...]
    h = jnp.where(h >= 0.0, h, 0.01 * h)
    o_ref[...] = jnp.dot(h, w2_ref[...], preferred_element_type=jnp.float32) + b2_ref[...]


def _tc_mlp(ra, rb, sa, sb, W1, b1, W2, b2):
    blk = 2048
    return pl.pallas_call(
        _mlp_body,
        grid=(_B // blk,),
        in_specs=[
            pl.BlockSpec((blk, _PACK), lambda i: (i, 0)),
            pl.BlockSpec((blk, _PACK), lambda i: (i, 0)),
            pl.BlockSpec((blk, 1), lambda i: (i, 0)),
            pl.BlockSpec((blk, 1), lambda i: (i, 0)),
            pl.BlockSpec((_D, _H), lambda i: (0, 0)),
            pl.BlockSpec((1, _H), lambda i: (0, 0)),
            pl.BlockSpec((_H, _N_CLASS), lambda i: (0, 0)),
            pl.BlockSpec((1, _N_CLASS), lambda i: (0, 0)),
        ],
        out_specs=pl.BlockSpec((blk, _N_CLASS), lambda i: (i, 0)),
        out_shape=jax.ShapeDtypeStruct((_B, _N_CLASS), jnp.float32),
    )(ra, rb, sa, sb, W1, b1.reshape(1, _H), W2, b2.reshape(1, _N_CLASS))


def kernel(x, table, W1, b1, W2, b2):
    xi = x.astype(jnp.int32)
    a = xi[:, 0]
    b = xi[:, 1]
    ga = (a >> 1).reshape(_NW, _NCH, _CH)
    gb = (b >> 1).reshape(_NW, _NCH, _CH)
    sa = (a & 1).reshape(_B, 1)
    sb = (b & 1).reshape(_B, 1)
    packed = table.reshape(500000, _PACK)
    ra, rb = _sc_gather(ga, gb, packed)
    return _tc_mlp(ra, rb, sa, sb, W1, b1, W2, b2)


# speedup vs baseline: 1.7119x; 1.7119x over previous
import functools

import jax
import jax.numpy as jnp
from jax import lax
from jax.experimental import pallas as pl
from jax.experimental.pallas import tpu as pltpu
from jax.experimental.pallas import tpu_sc as plsc

_B = 16384
_D = 64
_H = 64
_N_CLASS = 128
_E_FACTOR = 1.0

_NC = 2
_NS = 16
_NW = _NC * _NS
_L = 16
_BPW = _B // _NW
_CH = 128
_NCH = _BPW // _CH


def _gather_sum_body(a_hbm, b_hbm, table_hbm, out_hbm,
                     idx_va, idx_vb, rows_a, rows_b, sem):
    wid = lax.axis_index("s") * _NC + lax.axis_index("c")
    for j in range(_NCH):
        pltpu.sync_copy(a_hbm.at[wid, j], idx_va)
        pltpu.sync_copy(b_hbm.at[wid, j], idx_vb)

        def fire(g, carry):
            va = idx_va[pl.ds(g * _L, _L)]
            vb = idx_vb[pl.ds(g * _L, _L)]
            for k in range(_L):
                ia = va[k]
                ib = vb[k]
                r = g * _L + k
                pltpu.async_copy(table_hbm.at[pl.ds(ia, 1)], rows_a.at[pl.ds(r, 1)], sem)
                pltpu.async_copy(table_hbm.at[pl.ds(ib, 1)], rows_b.at[pl.ds(r, 1)], sem)
            return carry

        lax.fori_loop(0, _CH // _L, fire, 0)
        pltpu.make_async_copy(table_hbm.at[pl.ds(0, _CH)], rows_a, sem).wait()
        pltpu.make_async_copy(table_hbm.at[pl.ds(0, _CH)], rows_b, sem).wait()

        def add(r, carry):
            for c in range(_D // _L):
                sl = pl.ds(c * _L, _L)
                rows_a[r, sl] = rows_a[r, sl] + rows_b[r, sl]
            return carry

        lax.fori_loop(0, _CH, add, 0)
        base = (wid * _NCH + j) * _CH
        pltpu.sync_copy(rows_a, out_hbm.at[pl.ds(base, _CH)])


@jax.jit
def _sc_gather_sum(a_idx, b_idx, table):
    mesh = plsc.VectorSubcoreMesh(core_axis_name="c", subcore_axis_name="s")
    kern = functools.partial(
        pl.kernel,
        mesh=mesh,
        out_type=jax.ShapeDtypeStruct((_B, _D), jnp.float32),
        scratch_types=[
            pltpu.VMEM((_CH,), jnp.int32),
            pltpu.VMEM((_CH,), jnp.int32),
            pltpu.VMEM((_CH, _D), jnp.float32),
            pltpu.VMEM((_CH, _D), jnp.float32),
            pltpu.SemaphoreType.DMA,
        ],
    )(_gather_sum_body)
    return kern(a_idx, b_idx, table)


def _mlp_body(e_ref, w1_ref, b1_ref, w2_ref, b2_ref, o_ref):
    e = e_ref[...] * (1.0 / _E_FACTOR)
    h = jnp.dot(e, w1_ref[...], preferred_element_type=jnp.float32) + b1_ref[...]
    h = jnp.where(h >= 0.0, h, 0.01 * h)
    o_ref[...] = jnp.dot(h, w2_ref[...], preferred_element_type=jnp.float32) + b2_ref[...]


def _tc_mlp(e, W1, b1, W2, b2):
    blk = 2048
    return pl.pallas_call(
        _mlp_body,
        grid=(_B // blk,),
        in_specs=[
            pl.BlockSpec((blk, _D), lambda i: (i, 0)),
            pl.BlockSpec((_D, _H), lambda i: (0, 0)),
            pl.BlockSpec((1, _H), lambda i: (0, 0)),
            pl.BlockSpec((_H, _N_CLASS), lambda i: (0, 0)),
            pl.BlockSpec((1, _N_CLASS), lambda i: (0, 0)),
        ],
        out_specs=pl.BlockSpec((blk, _N_CLASS), lambda i: (i, 0)),
        out_shape=jax.ShapeDtypeStruct((_B, _N_CLASS), jnp.float32),
    )(e, W1, b1.reshape(1, _H), W2, b2.reshape(1, _N_CLASS))


def kernel(x, table, W1, b1, W2, b2):
    xi = x.astype(jnp.int32)
    a_idx = xi[:, 0].reshape(_NW, _NCH, _CH)
    b_idx = xi[:, 1].reshape(_NW, _NCH, _CH)
    e = _sc_gather_sum(a_idx, b_idx, table)
    return _tc_mlp(e, W1, b1, W2, b2)
